# baseline (speedup 1.0000x reference)
import functools

import jax
import jax.numpy as jnp
from jax import lax
from jax.experimental import pallas as pl
from jax.experimental.pallas import tpu as pltpu
from jax.experimental.pallas import tpu_sc as plsc

N = 10000
D = 128
E_RAW = 320000
E_TOT = E_RAW + N
NC, NS = 2, 16
NW = NC * NS

LANE = 128
ROWS_PER_TILE = 88
E_PAD = NW * ROWS_PER_TILE * LANE
ROWS_PER_CORE = (NW // NC) * ROWS_PER_TILE

PAD_ROWS = 10112
DUMMY = N
ZCHUNK = PAD_ROWS // NS

_mesh = plsc.VectorSubcoreMesh(core_axis_name="c", subcore_axis_name="s")


@functools.partial(
    pl.kernel,
    out_type=jax.ShapeDtypeStruct((NC, PAD_ROWS, 16), jnp.float32),
    mesh=_mesh,
    scratch_types=[
        pltpu.MemorySpace.VMEM_SHARED((PAD_ROWS, 16), jnp.float32),
        pltpu.MemorySpace.VMEM((ROWS_PER_TILE, LANE), jnp.int32),
        pltpu.MemorySpace.VMEM((LANE, 16), jnp.float32),
    ],
)
def _sc_degree(dst_rows, zeros16, ones16, out, deg_sh, idx_v, ones_v):
    cid = lax.axis_index("c")
    sid = lax.axis_index("s")
    rowbase = cid * ROWS_PER_CORE + sid * ROWS_PER_TILE
    pltpu.sync_copy(zeros16.at[pl.ds(sid * ZCHUNK, ZCHUNK)],
                    deg_sh.at[pl.ds(sid * ZCHUNK, ZCHUNK)])
    pltpu.sync_copy(dst_rows.at[pl.ds(rowbase, ROWS_PER_TILE)], idx_v)
    pltpu.sync_copy(ones16, ones_v)
    plsc.subcore_barrier()

    def body(j, carry):
        pltpu.sync_copy(ones_v, deg_sh.at[idx_v.at[j]], add=True)
        return carry

    lax.fori_loop(0, ROWS_PER_TILE, body, 0)
    plsc.subcore_barrier()
    pltpu.sync_copy(deg_sh.at[pl.ds(sid * ZCHUNK, ZCHUNK)],
                    out.at[cid].at[pl.ds(sid * ZCHUNK, ZCHUNK)])


@functools.partial(
    pl.kernel,
    out_type=jax.ShapeDtypeStruct((E_PAD, D), jnp.float32),
    mesh=_mesh,
    scratch_types=[
        pltpu.MemorySpace.VMEM_SHARED((PAD_ROWS, D), jnp.float32),
        pltpu.MemorySpace.VMEM((ROWS_PER_TILE, LANE), jnp.int32),
        pltpu.MemorySpace.VMEM((LANE, D), jnp.float32),
        pltpu.MemorySpace.VMEM((LANE, D), jnp.float32),
        pltpu.SemaphoreType.DMA,
        pltpu.SemaphoreType.DMA,
        pltpu.SemaphoreType.DMA,
        pltpu.SemaphoreType.DMA,
    ],
)
def _sc_gather_msgs(y, src_rows, msgs,
                    table_sh, src_v, rows_a, rows_b,
                    sem_ga, sem_gb, sem_wa, sem_wb):
    cid = lax.axis_index("c")
    sid = lax.axis_index("s")
    rowbase = cid * ROWS_PER_CORE + sid * ROWS_PER_TILE
    ebase = rowbase * LANE
    chunk = pl.ds(sid * ZCHUNK, ZCHUNK)
    pltpu.sync_copy(y.at[chunk], table_sh.at[chunk])
    pltpu.sync_copy(src_rows.at[pl.ds(rowbase, ROWS_PER_TILE)], src_v)
    plsc.subcore_barrier()

    half = ROWS_PER_TILE // 2
    pltpu.async_copy(table_sh.at[src_v.at[0]], rows_a, sem_ga)

    def body(k, carry):
        j0 = 2 * k
        j1 = j0 + 1
        pltpu.make_async_copy(table_sh.at[src_v.at[j0]],
                              rows_a, sem_ga).wait()

        @pl.when(k > 0)
        def _():
            pltpu.make_async_copy(rows_b, msgs.at[pl.ds(0, LANE)],
                                  sem_wb).wait()

        pltpu.async_copy(table_sh.at[src_v.at[j1]], rows_b, sem_gb)
        pltpu.async_copy(rows_a, msgs.at[pl.ds(ebase + j0 * LANE, LANE)],
                         sem_wa)
        pltpu.make_async_copy(table_sh.at[src_v.at[j1]],
                              rows_b, sem_gb).wait()
        pltpu.make_async_copy(rows_a, msgs.at[pl.ds(0, LANE)],
                              sem_wa).wait()

        @pl.when(k < half - 1)
        def _():
            pltpu.async_copy(table_sh.at[src_v.at[j0 + 2]], rows_a, sem_ga)

        pltpu.async_copy(rows_b, msgs.at[pl.ds(ebase + j1 * LANE, LANE)],
                         sem_wb)
        return carry

    lax.fori_loop(0, half, body, 0)
    pltpu.make_async_copy(rows_b, msgs.at[pl.ds(0, LANE)], sem_wb).wait()


@functools.partial(
    pl.kernel,
    out_type=jax.ShapeDtypeStruct((NC, PAD_ROWS, D), jnp.float32),
    mesh=_mesh,
    scratch_types=[
        pltpu.MemorySpace.VMEM_SHARED((PAD_ROWS, D), jnp.float32),
        pltpu.MemorySpace.VMEM((ROWS_PER_TILE, LANE), jnp.int32),
        pltpu.MemorySpace.VMEM((LANE, D), jnp.float32),
        pltpu.MemorySpace.VMEM((LANE, D), jnp.float32),
        pltpu.SemaphoreType.DMA,
        pltpu.SemaphoreType.DMA,
    ],
)
def _sc_scatter_msgs(msgs, dst_rows, zeros, out,
                     agg_sh, dst_v, rows_a, rows_b, sem_a, sem_b):
    cid = lax.axis_index("c")
    sid = lax.axis_index("s")
    rowbase = cid * ROWS_PER_CORE + sid * ROWS_PER_TILE
    ebase = rowbase * LANE
    chunk = pl.ds(sid * ZCHUNK, ZCHUNK)
    pltpu.sync_copy(zeros.at[chunk], agg_sh.at[chunk])
    pltpu.sync_copy(dst_rows.at[pl.ds(rowbase, ROWS_PER_TILE)], dst_v)
    plsc.subcore_barrier()

    half = ROWS_PER_TILE // 2
    pltpu.async_copy(msgs.at[pl.ds(ebase, LANE)], rows_a, sem_a)

    def body(k, carry):
        j0 = 2 * k
        j1 = j0 + 1
        pltpu.async_copy(msgs.at[pl.ds(ebase + j1 * LANE, LANE)],
                         rows_b, sem_b)
        pltpu.make_async_copy(msgs.at[pl.ds(0, LANE)], rows_a, sem_a).wait()
        pltpu.sync_copy(rows_a, agg_sh.at[dst_v.at[j0]], add=True)

        @pl.when(k < half - 1)
        def _():
            pltpu.async_copy(msgs.at[pl.ds(ebase + (j0 + 2) * LANE, LANE)],
                             rows_a, sem_a)

        pltpu.make_async_copy(msgs.at[pl.ds(0, LANE)], rows_b, sem_b).wait()
        pltpu.sync_copy(rows_b, agg_sh.at[dst_v.at[j1]], add=True)
        return carry

    lax.fori_loop(0, half, body, 0)
    plsc.subcore_barrier()
    pltpu.sync_copy(agg_sh.at[chunk], out.at[cid].at[chunk])


_BLK = 1264
_GRID = PAD_ROWS // _BLK


def _row_spec(w, blk=_BLK):
    return pl.BlockSpec((blk, w), lambda i: (i, 0))


def _full_spec(h, w):
    return pl.BlockSpec((h, w), lambda i: (0, 0))


def _tc_pre_body(x_ref, d0_ref, d1_ref, w_ref, y_ref, dinv_ref):
    deg = d0_ref[...] + d1_ref[...]
    dinv = lax.rsqrt(jnp.maximum(deg, 1e-12))
    d1 = dinv[:, 0:1]
    y_ref[...] = jnp.dot(x_ref[...], w_ref[...],
                         preferred_element_type=jnp.float32) * d1
    dinv_ref[...] = dinv


def _tc_pre(x, d0, d1, W0):
    return pl.pallas_call(
        _tc_pre_body,
        grid=(_GRID,),
        in_specs=[_row_spec(D), _row_spec(16), _row_spec(16),
                  _full_spec(D, D)],
        out_specs=[_row_spec(D), _row_spec(16)],
        out_shape=[jax.ShapeDtypeStruct((PAD_ROWS, D), jnp.float32),
                   jax.ShapeDtypeStruct((PAD_ROWS, 16), jnp.float32)],
    )(x, d0, d1, W0)


def _tc_mid_body(p0_ref, p1_ref, dinv_ref, b_ref, w_ref, y_ref):
    d1 = dinv_ref[:, 0:1]
    t = jnp.maximum((p0_ref[...] + p1_ref[...]) * d1 + b_ref[...], 0.0)
    y_ref[...] = jnp.dot(t, w_ref[...],
                         preferred_element_type=jnp.float32) * d1


def _tc_mid(p, dinv16, b, W):
    return pl.pallas_call(
        _tc_mid_body,
        grid=(_GRID,),
        in_specs=[_row_spec(D), _row_spec(D), _row_spec(16),
                  _full_spec(1, D), _full_spec(D, D)],
        out_specs=_row_spec(D),
        out_shape=jax.ShapeDtypeStruct((PAD_ROWS, D), jnp.float32),
    )(p[0], p[1], dinv16, b, W)


def _tc_final_body(p0_ref, p1_ref, dinv_ref, b_ref, wl_ref, bl_ref, o_ref):
    d1 = dinv_ref[:, 0:1]
    t = jnp.maximum((p0_ref[...] + p1_ref[...]) * d1 + b_ref[...], 0.0)
    o_ref[...] = jnp.dot(t, wl_ref[...],
                         preferred_element_type=jnp.float32) + bl_ref[...]


def _tc_final(p, dinv16, b, Wl, bl):
    fb = 2000

    def fspec(w):
        return pl.BlockSpec((fb, w), lambda i: (i, 0))

    return pl.pallas_call(
        _tc_final_body,
        grid=(N // fb,),
        in_specs=[fspec(D), fspec(D), fspec(16), _full_spec(1, D),
                  _full_spec(D, 1), _full_spec(1, 1)],
        out_specs=fspec(1),
        out_shape=jax.ShapeDtypeStruct((N, 1), jnp.float32),
    )(p[0], p[1], dinv16, b, Wl, bl)


def _edge_pass(y, src_rows, dst_rows, zeros):
    msgs = _sc_gather_msgs(y, src_rows)
    return _sc_scatter_msgs(msgs, dst_rows, zeros)


def kernel(x, edge_index, W0, b0, W1, b1, W2, b2, Wl, bl):
    loops = jnp.arange(N, dtype=jnp.int32)
    src = jnp.concatenate([edge_index[0].astype(jnp.int32), loops])
    dst = jnp.concatenate([edge_index[1].astype(jnp.int32), loops])
    pad = E_PAD - E_TOT
    src = jnp.concatenate([src, jnp.zeros((pad,), jnp.int32)])
    dst = jnp.concatenate([dst, jnp.full((pad,), DUMMY, jnp.int32)])
    src_rows = src.reshape(-1, LANE)
    dst_rows = dst.reshape(-1, LANE)

    zeros16 = jnp.zeros((PAD_ROWS, 16), jnp.float32)
    ones16 = jnp.ones((LANE, 16), jnp.float32)
    zeros = jnp.zeros((PAD_ROWS, D), jnp.float32)

    degp = _sc_degree(dst_rows, zeros16, ones16)
    y, dinv16 = _tc_pre(x, degp[0], degp[1], W0)

    p = _edge_pass(y, src_rows, dst_rows, zeros)
    y = _tc_mid(p, dinv16, b0.reshape(1, D), W1)

    p = _edge_pass(y, src_rows, dst_rows, zeros)
    y = _tc_mid(p, dinv16, b1.reshape(1, D), W2)

    p = _edge_pass(y, src_rows, dst_rows, zeros)
    return _tc_final(p, dinv16, b2.reshape(1, D), Wl, bl.reshape(1, 1))


# speedup vs baseline: 1.0099x; 1.0099x over previous
import functools

import jax
import jax.numpy as jnp
from jax import lax
from jax.experimental import pallas as pl
from jax.experimental.pallas import tpu as pltpu
from jax.experimental.pallas import tpu_sc as plsc

N = 10000
D = 128
E_RAW = 320000
E_TOT = E_RAW + N
NC, NS = 2, 16
NW = NC * NS

LANE = 128
ROWS_PER_TILE = 88
E_PAD = NW * ROWS_PER_TILE * LANE
ROWS_PER_CORE = (NW // NC) * ROWS_PER_TILE

PAD_ROWS = 10112
DUMMY = N
ZCHUNK = PAD_ROWS // NS

_mesh = plsc.VectorSubcoreMesh(core_axis_name="c", subcore_axis_name="s")


@functools.partial(
    pl.kernel,
    out_type=jax.ShapeDtypeStruct((NC, PAD_ROWS, 16), jnp.float32),
    mesh=_mesh,
    scratch_types=[
        pltpu.MemorySpace.VMEM_SHARED((PAD_ROWS, 16), jnp.float32),
        pltpu.MemorySpace.VMEM((ROWS_PER_TILE, LANE), jnp.int32),
        pltpu.MemorySpace.VMEM((LANE, 16), jnp.float32),
        pltpu.SemaphoreType.DMA,
        pltpu.SemaphoreType.DMA,
    ],
)
def _sc_degree(dst_rows, zeros16, ones16, out, deg_sh, idx_v, ones_v,
               dsem_a, dsem_b):
    cid = lax.axis_index("c")
    sid = lax.axis_index("s")
    rowbase = cid * ROWS_PER_CORE + sid * ROWS_PER_TILE
    pltpu.sync_copy(zeros16.at[pl.ds(sid * ZCHUNK, ZCHUNK)],
                    deg_sh.at[pl.ds(sid * ZCHUNK, ZCHUNK)])
    pltpu.sync_copy(dst_rows.at[pl.ds(rowbase, ROWS_PER_TILE)], idx_v)
    pltpu.sync_copy(ones16, ones_v)
    plsc.subcore_barrier()

    half = ROWS_PER_TILE // 2
    pltpu.async_copy(ones_v, deg_sh.at[idx_v.at[0]], dsem_a, add=True)

    def body(k, carry):
        j0 = 2 * k
        j1 = j0 + 1
        pltpu.async_copy(ones_v, deg_sh.at[idx_v.at[j1]], dsem_b, add=True)
        pltpu.make_async_copy(ones_v, deg_sh.at[idx_v.at[j0]],
                              dsem_a).wait()

        @pl.when(k < half - 1)
        def _():
            pltpu.async_copy(ones_v, deg_sh.at[idx_v.at[j0 + 2]],
                             dsem_a, add=True)

        pltpu.make_async_copy(ones_v, deg_sh.at[idx_v.at[j1]],
                              dsem_b).wait()
        return carry

    lax.fori_loop(0, half, body, 0)
    plsc.subcore_barrier()
    pltpu.sync_copy(deg_sh.at[pl.ds(sid * ZCHUNK, ZCHUNK)],
                    out.at[cid].at[pl.ds(sid * ZCHUNK, ZCHUNK)])


@functools.partial(
    pl.kernel,
    out_type=jax.ShapeDtypeStruct((NC, PAD_ROWS, D), jnp.float32),
    mesh=_mesh,
    scratch_types=[
        pltpu.MemorySpace.HBM((E_PAD, D), jnp.float32),
        pltpu.MemorySpace.VMEM_SHARED((PAD_ROWS, D), jnp.float32),
        pltpu.MemorySpace.VMEM((ROWS_PER_TILE, LANE), jnp.int32),
        pltpu.MemorySpace.VMEM((LANE, D), jnp.float32),
        pltpu.MemorySpace.VMEM((LANE, D), jnp.float32),
        pltpu.SemaphoreType.DMA,
        pltpu.SemaphoreType.DMA,
        pltpu.SemaphoreType.DMA,
        pltpu.SemaphoreType.DMA,
    ],
)
def _sc_edge_pass(y, src_rows, dst_rows, zeros, out,
                  msgs, shmem, idx_v, rows_a, rows_b,
                  sem_ga, sem_gb, sem_wa, sem_wb):
    cid = lax.axis_index("c")
    sid = lax.axis_index("s")
    rowbase = cid * ROWS_PER_CORE + sid * ROWS_PER_TILE
    ebase = rowbase * LANE
    chunk = pl.ds(sid * ZCHUNK, ZCHUNK)
    half = ROWS_PER_TILE // 2

    pltpu.sync_copy(y.at[chunk], shmem.at[chunk])
    pltpu.sync_copy(src_rows.at[pl.ds(rowbase, ROWS_PER_TILE)], idx_v)
    plsc.subcore_barrier()

    pltpu.async_copy(shmem.at[idx_v.at[0]], rows_a, sem_ga)

    def body_a(k, carry):
        j0 = 2 * k
        j1 = j0 + 1
        pltpu.make_async_copy(shmem.at[idx_v.at[j0]], rows_a, sem_ga).wait()

        @pl.when(k > 0)
        def _():
            pltpu.make_async_copy(rows_b, msgs.at[pl.ds(0, LANE)],
                                  sem_wb).wait()

        pltpu.async_copy(shmem.at[idx_v.at[j1]], rows_b, sem_gb)
        pltpu.async_copy(rows_a, msgs.at[pl.ds(ebase + j0 * LANE, LANE)],
                         sem_wa)
        pltpu.make_async_copy(shmem.at[idx_v.at[j1]], rows_b, sem_gb).wait()
        pltpu.make_async_copy(rows_a, msgs.at[pl.ds(0, LANE)],
                              sem_wa).wait()

        @pl.when(k < half - 1)
        def _():
            pltpu.async_copy(shmem.at[idx_v.at[j0 + 2]], rows_a, sem_ga)

        pltpu.async_copy(rows_b, msgs.at[pl.ds(ebase + j1 * LANE, LANE)],
                         sem_wb)
        return carry

    lax.fori_loop(0, half, body_a, 0)
    pltpu.make_async_copy(rows_b, msgs.at[pl.ds(0, LANE)], sem_wb).wait()
    plsc.subcore_barrier()

    pltpu.sync_copy(zeros.at[chunk], shmem.at[chunk])
    pltpu.sync_copy(dst_rows.at[pl.ds(rowbase, ROWS_PER_TILE)], idx_v)
    plsc.subcore_barrier()

    pltpu.async_copy(msgs.at[pl.ds(ebase, LANE)], rows_a, sem_ga)

    def body_b(k, carry):
        j0 = 2 * k
        j1 = j0 + 1
        pltpu.async_copy(msgs.at[pl.ds(ebase + j1 * LANE, LANE)],
                         rows_b, sem_gb)
        pltpu.make_async_copy(msgs.at[pl.ds(0, LANE)], rows_a, sem_ga).wait()
        pltpu.sync_copy(rows_a, shmem.at[idx_v.at[j0]], add=True)

        @pl.when(k < half - 1)
        def _():
            pltpu.async_copy(msgs.at[pl.ds(ebase + (j0 + 2) * LANE, LANE)],
                             rows_a, sem_ga)

        pltpu.make_async_copy(msgs.at[pl.ds(0, LANE)], rows_b, sem_gb).wait()
        pltpu.sync_copy(rows_b, shmem.at[idx_v.at[j1]], add=True)
        return carry

    lax.fori_loop(0, half, body_b, 0)
    plsc.subcore_barrier()
    pltpu.sync_copy(shmem.at[chunk], out.at[cid].at[chunk])


_BLK = 1264
_GRID = PAD_ROWS // _BLK


def _row_spec(w, blk=_BLK):
    return pl.BlockSpec((blk, w), lambda i: (i, 0))


def _full_spec(h, w):
    return pl.BlockSpec((h, w), lambda i: (0, 0))


def _tc_pre_body(x_ref, d0_ref, d1_ref, w_ref, y_ref, dinv_ref):
    deg = d0_ref[...] + d1_ref[...]
    dinv = lax.rsqrt(jnp.maximum(deg, 1e-12))
    d1 = dinv[:, 0:1]
    y_ref[...] = jnp.dot(x_ref[...], w_ref[...],
                         preferred_element_type=jnp.float32) * d1
    dinv_ref[...] = dinv


def _tc_pre(x, d0, d1, W0):
    return pl.pallas_call(
        _tc_pre_body,
        grid=(_GRID,),
        in_specs=[_row_spec(D), _row_spec(16), _row_spec(16),
                  _full_spec(D, D)],
        out_specs=[_row_spec(D), _row_spec(16)],
        out_shape=[jax.ShapeDtypeStruct((PAD_ROWS, D), jnp.float32),
                   jax.ShapeDtypeStruct((PAD_ROWS, 16), jnp.float32)],
    )(x, d0, d1, W0)


def _tc_mid_body(p0_ref, p1_ref, dinv_ref, b_ref, w_ref, y_ref):
    d1 = dinv_ref[:, 0:1]
    t = jnp.maximum((p0_ref[...] + p1_ref[...]) * d1 + b_ref[...], 0.0)
    y_ref[...] = jnp.dot(t, w_ref[...],
                         preferred_element_type=jnp.float32) * d1


def _tc_mid(p, dinv16, b, W):
    return pl.pallas_call(
        _tc_mid_body,
        grid=(_GRID,),
        in_specs=[_row_spec(D), _row_spec(D), _row_spec(16),
                  _full_spec(1, D), _full_spec(D, D)],
        out_specs=_row_spec(D),
        out_shape=jax.ShapeDtypeStruct((PAD_ROWS, D), jnp.float32),
    )(p[0], p[1], dinv16, b, W)


def _tc_final_body(p0_ref, p1_ref, dinv_ref, b_ref, wl_ref, bl_ref, o_ref):
    d1 = dinv_ref[:, 0:1]
    t = jnp.maximum((p0_ref[...] + p1_ref[...]) * d1 + b_ref[...], 0.0)
    o_ref[...] = jnp.dot(t, wl_ref[...],
                         preferred_element_type=jnp.float32) + bl_ref[...]


def _tc_final(p, dinv16, b, Wl, bl):
    fb = 2000

    def fspec(w):
        return pl.BlockSpec((fb, w), lambda i: (i, 0))

    return pl.pallas_call(
        _tc_final_body,
        grid=(N // fb,),
        in_specs=[fspec(D), fspec(D), fspec(16), _full_spec(1, D),
                  _full_spec(D, 1), _full_spec(1, 1)],
        out_specs=fspec(1),
        out_shape=jax.ShapeDtypeStruct((N, 1), jnp.float32),
    )(p[0], p[1], dinv16, b, Wl, bl)


def kernel(x, edge_index, W0, b0, W1, b1, W2, b2, Wl, bl):
    loops = jnp.arange(N, dtype=jnp.int32)
    src = jnp.concatenate([edge_index[0].astype(jnp.int32), loops])
    dst = jnp.concatenate([edge_index[1].astype(jnp.int32), loops])
    pad = E_PAD - E_TOT
    src = jnp.concatenate([src, jnp.zeros((pad,), jnp.int32)])
    dst = jnp.concatenate([dst, jnp.full((pad,), DUMMY, jnp.int32)])
    src_rows = src.reshape(-1, LANE)
    dst_rows = dst.reshape(-1, LANE)

    zeros16 = jnp.zeros((PAD_ROWS, 16), jnp.float32)
    ones16 = jnp.ones((LANE, 16), jnp.float32)
    zeros = jnp.zeros((PAD_ROWS, D), jnp.float32)

    degp = _sc_degree(dst_rows, zeros16, ones16)
    y, dinv16 = _tc_pre(x, degp[0], degp[1], W0)

    p = _sc_edge_pass(y, src_rows, dst_rows, zeros)
    y = _tc_mid(p, dinv16, b0.reshape(1, D), W1)

    p = _sc_edge_pass(y, src_rows, dst_rows, zeros)
    y = _tc_mid(p, dinv16, b1.reshape(1, D), W2)

    p = _sc_edge_pass(y, src_rows, dst_rows, zeros)
    return _tc_final(p, dinv16, b2.reshape(1, D), Wl, bl.reshape(1, 1))


# speedup vs baseline: 1.1159x; 1.1049x over previous
import functools

import jax
import jax.numpy as jnp
from jax import lax
from jax.experimental import pallas as pl
from jax.experimental.pallas import tpu as pltpu
from jax.experimental.pallas import tpu_sc as plsc

N = 10000
D = 128
E_RAW = 320000
E_TOT = E_RAW + N
NC, NS = 2, 16
NW = NC * NS

LANE = 128
ROWS_PER_TILE = 82
E_PAD = NW * ROWS_PER_TILE * LANE
ROWS_PER_CORE = (NW // NC) * ROWS_PER_TILE

PAD_ROWS = 10112
DUMMY = N
ZCHUNK = PAD_ROWS // NS

_mesh = plsc.VectorSubcoreMesh(core_axis_name="c", subcore_axis_name="s")


@functools.partial(
    pl.kernel,
    out_type=jax.ShapeDtypeStruct((NC, PAD_ROWS, 16), jnp.float32),
    mesh=_mesh,
    scratch_types=[
        pltpu.MemorySpace.VMEM_SHARED((PAD_ROWS, 16), jnp.float32),
        pltpu.MemorySpace.VMEM((ROWS_PER_TILE, LANE), jnp.int32),
        pltpu.MemorySpace.VMEM((LANE, 16), jnp.float32),
        pltpu.SemaphoreType.DMA,
        pltpu.SemaphoreType.DMA,
    ],
)
def _sc_degree(dst_rows, zeros16, ones16, out, deg_sh, idx_v, ones_v,
               dsem_a, dsem_b):
    cid = lax.axis_index("c")
    sid = lax.axis_index("s")
    w = cid * NS + sid
    pltpu.sync_copy(zeros16.at[pl.ds(sid * ZCHUNK, ZCHUNK)],
                    deg_sh.at[pl.ds(sid * ZCHUNK, ZCHUNK)])
    pltpu.sync_copy(dst_rows.at[w], idx_v)
    pltpu.sync_copy(ones16, ones_v)
    plsc.subcore_barrier()

    half = ROWS_PER_TILE // 2
    pltpu.async_copy(ones_v, deg_sh.at[idx_v.at[0]], dsem_a, add=True)

    def body(k, carry):
        j0 = 2 * k
        j1 = j0 + 1
        pltpu.async_copy(ones_v, deg_sh.at[idx_v.at[j1]], dsem_b, add=True)
        pltpu.make_async_copy(ones_v, deg_sh.at[idx_v.at[j0]],
                              dsem_a).wait()

        @pl.when(k < half - 1)
        def _():
            pltpu.async_copy(ones_v, deg_sh.at[idx_v.at[j0 + 2]],
                             dsem_a, add=True)

        pltpu.make_async_copy(ones_v, deg_sh.at[idx_v.at[j1]],
                              dsem_b).wait()
        return carry

    lax.fori_loop(0, half, body, 0)
    plsc.subcore_barrier()
    pltpu.sync_copy(deg_sh.at[pl.ds(sid * ZCHUNK, ZCHUNK)],
                    out.at[cid].at[pl.ds(sid * ZCHUNK, ZCHUNK)])


@functools.partial(
    pl.kernel,
    out_type=jax.ShapeDtypeStruct((NC, PAD_ROWS, D), jnp.float32),
    mesh=_mesh,
    scratch_types=[
        pltpu.MemorySpace.HBM((E_PAD, D), jnp.float32),
        pltpu.MemorySpace.VMEM_SHARED((PAD_ROWS, D), jnp.float32),
        pltpu.MemorySpace.VMEM((ROWS_PER_TILE, LANE), jnp.int32),
        pltpu.MemorySpace.VMEM((LANE, D), jnp.float32),
        pltpu.MemorySpace.VMEM((LANE, D), jnp.float32),
        pltpu.SemaphoreType.DMA,
        pltpu.SemaphoreType.DMA,
        pltpu.SemaphoreType.DMA,
        pltpu.SemaphoreType.DMA,
    ],
)
def _sc_edge_pass(y, src_rows, dst_rows, zeros, out,
                  msgs, shmem, idx_v, rows_a, rows_b,
                  sem_ga, sem_gb, sem_wa, sem_wb):
    cid = lax.axis_index("c")
    sid = lax.axis_index("s")
    w = cid * NS + sid
    ebase = w * ROWS_PER_TILE * LANE
    chunk = pl.ds(sid * ZCHUNK, ZCHUNK)
    half = ROWS_PER_TILE // 2

    pltpu.sync_copy(y.at[chunk], shmem.at[chunk])
    pltpu.sync_copy(src_rows.at[w], idx_v)
    plsc.subcore_barrier()

    pltpu.async_copy(shmem.at[idx_v.at[0]], rows_a, sem_ga)

    def body_a(k, carry):
        j0 = 2 * k
        j1 = j0 + 1
        pltpu.make_async_copy(shmem.at[idx_v.at[j0]], rows_a, sem_ga).wait()

        @pl.when(k > 0)
        def _():
            pltpu.make_async_copy(rows_b, msgs.at[pl.ds(0, LANE)],
                                  sem_wb).wait()

        pltpu.async_copy(shmem.at[idx_v.at[j1]], rows_b, sem_gb)
        pltpu.async_copy(rows_a, msgs.at[pl.ds(ebase + j0 * LANE, LANE)],
                         sem_wa)
        pltpu.make_async_copy(shmem.at[idx_v.at[j1]], rows_b, sem_gb).wait()
        pltpu.make_async_copy(rows_a, msgs.at[pl.ds(0, LANE)],
                              sem_wa).wait()

        @pl.when(k < half - 1)
        def _():
            pltpu.async_copy(shmem.at[idx_v.at[j0 + 2]], rows_a, sem_ga)

        pltpu.async_copy(rows_b, msgs.at[pl.ds(ebase + j1 * LANE, LANE)],
                         sem_wb)
        return carry

    lax.fori_loop(0, half, body_a, 0)
    pltpu.make_async_copy(rows_b, msgs.at[pl.ds(0, LANE)], sem_wb).wait()
    plsc.subcore_barrier()

    pltpu.sync_copy(zeros.at[chunk], shmem.at[chunk])
    pltpu.sync_copy(dst_rows.at[w], idx_v)
    plsc.subcore_barrier()

    pltpu.async_copy(msgs.at[pl.ds(ebase, LANE)], rows_a, sem_ga)

    def body_b(k, carry):
        j0 = 2 * k
        j1 = j0 + 1
        pltpu.async_copy(msgs.at[pl.ds(ebase + j1 * LANE, LANE)],
                         rows_b, sem_gb)
        pltpu.make_async_copy(msgs.at[pl.ds(0, LANE)], rows_a, sem_ga).wait()
        pltpu.sync_copy(rows_a, shmem.at[idx_v.at[j0]], add=True)

        @pl.when(k < half - 1)
        def _():
            pltpu.async_copy(msgs.at[pl.ds(ebase + (j0 + 2) * LANE, LANE)],
                             rows_a, sem_ga)

        pltpu.make_async_copy(msgs.at[pl.ds(0, LANE)], rows_b, sem_gb).wait()
        pltpu.sync_copy(rows_b, shmem.at[idx_v.at[j1]], add=True)
        return carry

    lax.fori_loop(0, half, body_b, 0)
    plsc.subcore_barrier()
    pltpu.sync_copy(shmem.at[chunk], out.at[cid].at[chunk])


_BLK = 1264
_GRID = PAD_ROWS // _BLK


def _row_spec(w, blk=_BLK):
    return pl.BlockSpec((blk, w), lambda i: (i, 0))


def _full_spec(h, w):
    return pl.BlockSpec((h, w), lambda i: (0, 0))


def _tc_pre_body(x_ref, d0_ref, d1_ref, w_ref, y_ref, dinv_ref):
    deg = d0_ref[...] + d1_ref[...]
    dinv = lax.rsqrt(jnp.maximum(deg, 1e-12))
    d1 = dinv[:, 0:1]
    y_ref[...] = jnp.dot(x_ref[...], w_ref[...],
                         preferred_element_type=jnp.float32) * d1
    dinv_ref[...] = dinv


def _tc_pre(x, d0, d1, W0):
    return pl.pallas_call(
        _tc_pre_body,
        grid=(_GRID,),
        in_specs=[_row_spec(D), _row_spec(16), _row_spec(16),
                  _full_spec(D, D)],
        out_specs=[_row_spec(D), _row_spec(16)],
        out_shape=[jax.ShapeDtypeStruct((PAD_ROWS, D), jnp.float32),
                   jax.ShapeDtypeStruct((PAD_ROWS, 16), jnp.float32)],
    )(x, d0, d1, W0)


def _tc_mid_body(p0_ref, p1_ref, dinv_ref, b_ref, w_ref, y_ref):
    d1 = dinv_ref[:, 0:1]
    t = jnp.maximum((p0_ref[...] + p1_ref[...]) * d1 + b_ref[...], 0.0)
    y_ref[...] = jnp.dot(t, w_ref[...],
                         preferred_element_type=jnp.float32) * d1


def _tc_mid(p, dinv16, b, W):
    return pl.pallas_call(
        _tc_mid_body,
        grid=(_GRID,),
        in_specs=[_row_spec(D), _row_spec(D), _row_spec(16),
                  _full_spec(1, D), _full_spec(D, D)],
        out_specs=_row_spec(D),
        out_shape=jax.ShapeDtypeStruct((PAD_ROWS, D), jnp.float32),
    )(p[0], p[1], dinv16, b, W)


def _tc_final_body(p0_ref, p1_ref, dinv_ref, b_ref, wl_ref, bl_ref, o_ref):
    d1 = dinv_ref[:, 0:1]
    t = jnp.maximum((p0_ref[...] + p1_ref[...]) * d1 + b_ref[...], 0.0)
    o_ref[...] = jnp.dot(t, wl_ref[...],
                         preferred_element_type=jnp.float32) + bl_ref[...]


def _tc_final(p, dinv16, b, Wl, bl):
    fb = 2000

    def fspec(w):
        return pl.BlockSpec((fb, w), lambda i: (i, 0))

    return pl.pallas_call(
        _tc_final_body,
        grid=(N // fb,),
        in_specs=[fspec(D), fspec(D), fspec(16), _full_spec(1, D),
                  _full_spec(D, 1), _full_spec(1, 1)],
        out_specs=fspec(1),
        out_shape=jax.ShapeDtypeStruct((N, 1), jnp.float32),
    )(p[0], p[1], dinv16, b, Wl, bl)


def kernel(x, edge_index, W0, b0, W1, b1, W2, b2, Wl, bl):
    loops = jnp.arange(N, dtype=jnp.int32)
    src = jnp.concatenate([edge_index[0].astype(jnp.int32), loops])
    dst = jnp.concatenate([edge_index[1].astype(jnp.int32), loops])
    pad = E_PAD - E_TOT
    src = jnp.concatenate([src, jnp.zeros((pad,), jnp.int32)])
    dst = jnp.concatenate([dst, jnp.full((pad,), DUMMY, jnp.int32)])
    src_rows = src.reshape(NW, ROWS_PER_TILE, LANE)
    dst_rows = dst.reshape(NW, ROWS_PER_TILE, LANE)

    zeros16 = jnp.zeros((PAD_ROWS, 16), jnp.float32)
    ones16 = jnp.ones((LANE, 16), jnp.float32)
    zeros = jnp.zeros((PAD_ROWS, D), jnp.float32)

    degp = _sc_degree(dst_rows, zeros16, ones16)
    y, dinv16 = _tc_pre(x, degp[0], degp[1], W0)

    p = _sc_edge_pass(y, src_rows, dst_rows, zeros)
    y = _tc_mid(p, dinv16, b0.reshape(1, D), W1)

    p = _sc_edge_pass(y, src_rows, dst_rows, zeros)
    y = _tc_mid(p, dinv16, b1.reshape(1, D), W2)

    p = _sc_edge_pass(y, src_rows, dst_rows, zeros)
    return _tc_final(p, dinv16, b2.reshape(1, D), Wl, bl.reshape(1, 1))
